# baseline (speedup 1.0000x reference)
import functools

import jax
import jax.numpy as jnp
from jax import lax
from jax.experimental import pallas as pl
from jax.experimental.pallas import tpu as pltpu
from jax.experimental.pallas import tpu_sc as plsc

N = 10000
NP = 10240
D = 128
E = 320000
EP = 327680
CHUNKS = 160
ROWS_PT = NP // 16
EPS = 1e-10


def _atanh(z):
    return 0.5 * jnp.log((1.0 + z) / (1.0 - z))


def _log_of_exp(v):
    n = jnp.sqrt(jnp.sum(v * v, axis=1, keepdims=True))
    nc = jnp.maximum(n, EPS)
    y = jnp.tanh(nc) * v / nc
    ny = jnp.sqrt(jnp.sum(y * y, axis=1, keepdims=True))
    ncl = jnp.clip(ny, EPS, 1.0 - 1e-5)
    nd = jnp.maximum(ny, EPS)
    return _atanh(ncl) * y / nd


def _exp_map(v):
    n = jnp.sqrt(jnp.sum(v * v, axis=1, keepdims=True))
    nc = jnp.maximum(n, EPS)
    return jnp.tanh(nc) * v / nc


def _feats_body(x0h, x1h, x2h, e0h, e1h, e2h, f0h, f1h, f2h,
                i0, i1, i2, r0, r1, r2, sem):
    cid = lax.axis_index("c")
    sid = lax.axis_index("s")
    wid = sid * 2 + cid
    base = wid * 320
    pltpu.sync_copy(x0h.at[pl.ds(base, 320)], i0)
    pltpu.sync_copy(x1h.at[pl.ds(base, 320)], i1)
    pltpu.sync_copy(x2h.at[pl.ds(base, 320)], i2)
    for c in range(4):
        o = c * 80
        d0 = pltpu.async_copy(e0h.at[i0.at[pl.ds(o, 80)]], r0, sem)
        d1 = pltpu.async_copy(e1h.at[i1.at[pl.ds(o, 80)]], r1, sem)
        d2 = pltpu.async_copy(e2h.at[i2.at[pl.ds(o, 80)]], r2, sem)
        d0.wait()
        d1.wait()
        d2.wait()
        pltpu.sync_copy(r0, f0h.at[pl.ds(base + o, 80)])
        pltpu.sync_copy(r1, f1h.at[pl.ds(base + o, 80)])
        pltpu.sync_copy(r2, f2h.at[pl.ds(base + o, 80)])


def _gather_feats(x0, x1, x2, emb0, emb1, emb2):
    mesh = plsc.VectorSubcoreMesh(core_axis_name="c", subcore_axis_name="s")
    fn = pl.kernel(
        _feats_body,
        out_type=(jax.ShapeDtypeStruct((NP, 64), jnp.float32),
                  jax.ShapeDtypeStruct((NP, 32), jnp.float32),
                  jax.ShapeDtypeStruct((NP, 32), jnp.float32)),
        mesh=mesh,
        scratch_types=[
            pltpu.VMEM((320,), jnp.int32),
            pltpu.VMEM((320,), jnp.int32),
            pltpu.VMEM((320,), jnp.int32),
            pltpu.VMEM((80, 64), jnp.float32),
            pltpu.VMEM((80, 32), jnp.float32),
            pltpu.VMEM((80, 32), jnp.float32),
            pltpu.SemaphoreType.DMA,
        ],
        compiler_params=pltpu.CompilerParams(use_tc_tiling_on_sc=False),
    )
    return fn(x0, x1, x2, emb0, emb1, emb2)


SCH = 8
HD = 64


def _agg_body(logz_h, src_h, dst_h, z2_h, z1_h, agg_h, deg0_h, deg1_h,
              logz_sh, agg_sh, deg_sh, src_v, dst_v,
              r0, r1, ones_v, gsem, ssem, dsem):
    cid = lax.axis_index("c")
    sid = lax.axis_index("s")
    rslc = pl.ds(sid * ROWS_PT, ROWS_PT)
    for j in range(8):
        ones_v[pl.ds(j * 16, 16)] = jnp.ones((16,), jnp.float32)
    rows = (r0, r1)
    for p in range(2):
        pltpu.sync_copy(logz_h.at[cid, p, rslc], logz_sh.at[rslc])
        pltpu.sync_copy(z2_h, agg_sh.at[rslc])
        if p == 0:
            pltpu.sync_copy(z1_h, deg_sh.at[rslc])
        plsc.subcore_barrier()

        def stage(st, carry):
            base = sid * CHUNKS + st * SCH
            pltpu.sync_copy(src_h.at[cid, pl.ds(base, SCH)], src_v)
            pltpu.sync_copy(dst_h.at[cid, pl.ds(base, SCH)], dst_v)
            scat = [None, None]
            gath = [None, None]
            degs = []
            for j in range(SCH + 1):
                if j < SCH:
                    b = j % 2
                    if scat[b] is not None:
                        scat[b].wait()
                    gath[b] = pltpu.async_copy(logz_sh.at[src_v.at[j]],
                                               rows[b], gsem)
                    if p == 0:
                        degs.append(pltpu.async_copy(
                            ones_v, deg_sh.at[dst_v.at[j]], dsem, add=True))
                if j > 0:
                    q = (j - 1) % 2
                    gath[q].wait()
                    scat[q] = pltpu.async_copy(rows[q],
                                               agg_sh.at[dst_v.at[j - 1]],
                                               ssem, add=True)
            for s in scat:
                if s is not None:
                    s.wait()
            for dd in degs:
                dd.wait()
            return carry

        lax.fori_loop(0, CHUNKS // SCH, stage, 0)
        plsc.subcore_barrier()
        pltpu.sync_copy(agg_sh.at[rslc], agg_h.at[cid, p, rslc])
        if p == 0:
            @pl.when(cid == 0)
            def _():
                pltpu.sync_copy(deg_sh.at[rslc], deg0_h.at[rslc])

            @pl.when(cid == 1)
            def _():
                pltpu.sync_copy(deg_sh.at[rslc], deg1_h.at[rslc])


def _edge_agg(logz_s, src_s, dst_s, zeros2d, zeros1d):
    mesh = plsc.VectorSubcoreMesh(core_axis_name="c", subcore_axis_name="s")
    fn = pl.kernel(
        _agg_body,
        out_type=(jax.ShapeDtypeStruct((2, 2, NP, HD), jnp.float32),
                  jax.ShapeDtypeStruct((NP,), jnp.float32),
                  jax.ShapeDtypeStruct((NP,), jnp.float32)),
        mesh=mesh,
        scratch_types=[
            pltpu.VMEM_SHARED((NP, HD), jnp.float32),
            pltpu.VMEM_SHARED((NP, HD), jnp.float32),
            pltpu.VMEM_SHARED((NP,), jnp.float32),
            pltpu.VMEM((SCH, 128), jnp.int32),
            pltpu.VMEM((SCH, 128), jnp.int32),
            pltpu.VMEM((128, HD), jnp.float32),
            pltpu.VMEM((128, HD), jnp.float32),
            pltpu.VMEM((128,), jnp.float32),
            pltpu.SemaphoreType.DMA,
            pltpu.SemaphoreType.DMA,
            pltpu.SemaphoreType.DMA,
        ],
        compiler_params=pltpu.CompilerParams(use_tc_tiling_on_sc=True),
    )
    return fn(logz_s, src_s, dst_s, zeros2d, zeros1d)


def _dense1_body(f0_ref, f1_ref, f2_ref, w0_ref, w1_ref, w2_ref, b_ref, o_ref):
    hi = lax.Precision.HIGHEST
    v = (jnp.dot(f0_ref[...], w0_ref[0], preferred_element_type=jnp.float32,
                 precision=hi)
         + jnp.dot(f1_ref[...], w1_ref[0], preferred_element_type=jnp.float32,
                   precision=hi)
         + jnp.dot(f2_ref[...], w2_ref[0], preferred_element_type=jnp.float32,
                   precision=hi)
         + b_ref[0])
    lz = _log_of_exp(v)
    o_ref[0, 0] = lz[:, :HD]
    o_ref[0, 1] = lz[:, HD:]


def _dense1(f0, f1, f2, Wi_s, bi_s):
    grid = (2, NP // 1280)
    return pl.pallas_call(
        _dense1_body,
        grid=grid,
        in_specs=[
            pl.BlockSpec((1280, 64), lambda m, r: (r, 0)),
            pl.BlockSpec((1280, 32), lambda m, r: (r, 0)),
            pl.BlockSpec((1280, 32), lambda m, r: (r, 0)),
            pl.BlockSpec((1, 64, D), lambda m, r: (m, 0, 0)),
            pl.BlockSpec((1, 32, D), lambda m, r: (m, 0, 0)),
            pl.BlockSpec((1, 32, D), lambda m, r: (m, 0, 0)),
            pl.BlockSpec((1, 1, D), lambda m, r: (m, 0, 0)),
        ],
        out_specs=pl.BlockSpec((1, 2, 1280, HD), lambda m, r: (m, 0, r, 0)),
        out_shape=jax.ShapeDtypeStruct((2, 2, NP, HD), jnp.float32),
    )(f0, f1, f2, Wi_s[:, :64], Wi_s[:, 64:96], Wi_s[:, 96:], bi_s)


def _dense2_body(a_ref, l_ref, d_ref, w_ref, b_ref, o_ref):
    a = jnp.concatenate([a_ref[0, 0], a_ref[0, 1]], axis=1)
    lz = jnp.concatenate([l_ref[0, 0], l_ref[0, 1]], axis=1)
    dg = d_ref[0]
    m = (a + lz) / (dg + 1.0)
    u = _log_of_exp(m)
    z = jnp.dot(u, w_ref[0], preferred_element_type=jnp.float32,
                precision=lax.Precision.HIGHEST) + b_ref[0]
    o_ref[0] = _exp_map(z)


def _dense2(agg_s, logz_s, deg3, Wo_s, bo_s):
    grid = (2, NP // 1280)
    return pl.pallas_call(
        _dense2_body,
        grid=grid,
        in_specs=[
            pl.BlockSpec((1, 2, 1280, HD), lambda m, r: (m, 0, r, 0)),
            pl.BlockSpec((1, 2, 1280, HD), lambda m, r: (m, 0, r, 0)),
            pl.BlockSpec((1, 1280, 1), lambda m, r: (m, r, 0)),
            pl.BlockSpec((1, D, D), lambda m, r: (m, 0, 0)),
            pl.BlockSpec((1, 1, D), lambda m, r: (m, 0, 0)),
        ],
        out_specs=pl.BlockSpec((1, 1280, D), lambda m, r: (m, r, 0)),
        out_shape=jax.ShapeDtypeStruct((2, NP, D), jnp.float32),
    )(agg_s, logz_s, deg3, Wo_s, bo_s)


def _prep_edges(ei, mode):
    src = ei[0].astype(jnp.int32)
    dst = ei[1].astype(jnp.int32)
    srcp = jnp.concatenate(
        [src, jnp.zeros((EP - E,), jnp.int32)]).reshape(EP // 128, 128)
    dstp = jnp.concatenate(
        [dst, jnp.full((EP - E,), NP - 1, jnp.int32)]).reshape(EP // 128, 128)
    return srcp, dstp


def kernel(x, edge_index_click, edge_index_buy, emb0, emb1, emb2,
           W_in_click, b_in_click, W_out_click, b_out_click,
           W_in_buy, b_in_buy, W_out_buy, b_out_buy):
    xi = jnp.pad(x.astype(jnp.int32), ((0, NP - N), (0, 0)))
    x0, x1, x2 = xi[:, 0], xi[:, 1], xi[:, 2]

    sc, dc = _prep_edges(edge_index_click, 0)
    sb, db = _prep_edges(edge_index_buy, 1)
    src_s = jnp.stack([sc, sb])
    dst_s = jnp.stack([dc, db])

    Wi_s = jnp.stack([W_in_click, W_in_buy])
    bi_s = jnp.stack([b_in_click, b_in_buy]).reshape(2, 1, D)
    Wo_s = jnp.stack([W_out_click, W_out_buy])
    bo_s = jnp.stack([b_out_click, b_out_buy]).reshape(2, 1, D)

    zeros2d = jnp.zeros((ROWS_PT, HD), jnp.float32)
    zeros1d = jnp.zeros((ROWS_PT,), jnp.float32)

    f0, f1, f2 = _gather_feats(x0, x1, x2, emb0, emb1, emb2)
    logz_s = _dense1(f0, f1, f2, Wi_s, bi_s)
    agg_s, deg0, deg1 = _edge_agg(logz_s, src_s, dst_s, zeros2d, zeros1d)
    deg3 = jnp.stack([deg0, deg1]).reshape(2, NP, 1)
    out_s = _dense2(agg_s, logz_s, deg3, Wo_s, bo_s)
    return (out_s[0, :N], out_s[1, :N])


# speedup vs baseline: 1.2539x; 1.2539x over previous
import functools

import jax
import jax.numpy as jnp
from jax import lax
from jax.experimental import pallas as pl
from jax.experimental.pallas import tpu as pltpu
from jax.experimental.pallas import tpu_sc as plsc

N = 10000
NP = 10240
D = 128
E = 320000
EP = 327680
CHUNKS = 160
ROWS_PT = NP // 16
EPS = 1e-10


def _atanh(z):
    return 0.5 * jnp.log((1.0 + z) / (1.0 - z))


def _log_of_exp(v):
    n = jnp.sqrt(jnp.sum(v * v, axis=1, keepdims=True))
    nc = jnp.maximum(n, EPS)
    y = jnp.tanh(nc) * v / nc
    ny = jnp.sqrt(jnp.sum(y * y, axis=1, keepdims=True))
    ncl = jnp.clip(ny, EPS, 1.0 - 1e-5)
    nd = jnp.maximum(ny, EPS)
    return _atanh(ncl) * y / nd


def _exp_map(v):
    n = jnp.sqrt(jnp.sum(v * v, axis=1, keepdims=True))
    nc = jnp.maximum(n, EPS)
    return jnp.tanh(nc) * v / nc


def _feats_body(x0h, x1h, x2h, e0h, e1h, e2h, f0h, f1h, f2h,
                i0, i1, i2, r0, r1, r2, sem):
    cid = lax.axis_index("c")
    sid = lax.axis_index("s")
    wid = sid * 2 + cid
    base = wid * 320
    pltpu.sync_copy(x0h.at[pl.ds(base, 320)], i0)
    pltpu.sync_copy(x1h.at[pl.ds(base, 320)], i1)
    pltpu.sync_copy(x2h.at[pl.ds(base, 320)], i2)
    for c in range(4):
        o = c * 80
        d0 = pltpu.async_copy(e0h.at[i0.at[pl.ds(o, 80)]], r0, sem)
        d1 = pltpu.async_copy(e1h.at[i1.at[pl.ds(o, 80)]], r1, sem)
        d2 = pltpu.async_copy(e2h.at[i2.at[pl.ds(o, 80)]], r2, sem)
        d0.wait()
        d1.wait()
        d2.wait()
        pltpu.sync_copy(r0, f0h.at[pl.ds(base + o, 80)])
        pltpu.sync_copy(r1, f1h.at[pl.ds(base + o, 80)])
        pltpu.sync_copy(r2, f2h.at[pl.ds(base + o, 80)])


def _gather_feats(x0, x1, x2, emb0, emb1, emb2):
    mesh = plsc.VectorSubcoreMesh(core_axis_name="c", subcore_axis_name="s")
    fn = pl.kernel(
        _feats_body,
        out_type=(jax.ShapeDtypeStruct((NP, 64), jnp.float32),
                  jax.ShapeDtypeStruct((NP, 32), jnp.float32),
                  jax.ShapeDtypeStruct((NP, 32), jnp.float32)),
        mesh=mesh,
        scratch_types=[
            pltpu.VMEM((320,), jnp.int32),
            pltpu.VMEM((320,), jnp.int32),
            pltpu.VMEM((320,), jnp.int32),
            pltpu.VMEM((80, 64), jnp.float32),
            pltpu.VMEM((80, 32), jnp.float32),
            pltpu.VMEM((80, 32), jnp.float32),
            pltpu.SemaphoreType.DMA,
        ],
        compiler_params=pltpu.CompilerParams(use_tc_tiling_on_sc=False),
    )
    return fn(x0, x1, x2, emb0, emb1, emb2)


SCH = 32
HD = 64


def _agg_body(logz_h, src_h, dst_h, z2_h, z1_h, agg_h, deg_h,
              logz_sh, agg_sh, deg_sh, src_v, dst_v,
              r0, r1, r2, r3, ones_v, gsem, ssem, dsem):
    cid = lax.axis_index("c")
    sid = lax.axis_index("s")
    rslc = pl.ds(sid * ROWS_PT, ROWS_PT)
    for j in range(8):
        ones_v[pl.ds(j * 16, 16)] = jnp.ones((16,), jnp.float32)
    rows = (r0, r1, r2, r3)
    for p in range(2):
        pltpu.sync_copy(logz_h.at[cid, rslc, pl.ds(p * HD, HD)],
                        logz_sh.at[rslc])
        pltpu.sync_copy(z2_h, agg_sh.at[rslc])
        if p == 0:
            pltpu.sync_copy(z1_h, deg_sh.at[rslc])
        plsc.subcore_barrier()

        def stage(st, carry):
            base = sid * CHUNKS + st * SCH
            pltpu.sync_copy(src_h.at[cid, pl.ds(base, SCH)], src_v)
            pltpu.sync_copy(dst_h.at[cid, pl.ds(base, SCH)], dst_v)
            scat = [None, None, None, None]
            gath = [None, None, None, None]
            degs = []
            for j in range(SCH + 2):
                if j < SCH:
                    b = j & 3
                    if scat[b] is not None:
                        scat[b].wait()
                    gath[b] = pltpu.async_copy(logz_sh.at[src_v.at[j]],
                                               rows[b], gsem)
                    if p == 0:
                        degs.append(pltpu.async_copy(
                            ones_v, deg_sh.at[dst_v.at[j]], dsem, add=True))
                if j > 1:
                    q = (j - 2) & 3
                    gath[q].wait()
                    scat[q] = pltpu.async_copy(rows[q],
                                               agg_sh.at[dst_v.at[j - 2]],
                                               ssem, add=True)
            for s in scat:
                s.wait()
            for dd in degs:
                dd.wait()
            return carry

        lax.fori_loop(0, CHUNKS // SCH, stage, 0)
        plsc.subcore_barrier()
        pltpu.sync_copy(agg_sh.at[rslc],
                        agg_h.at[cid, rslc, pl.ds(p * HD, HD)])
        if p == 0:
            pltpu.sync_copy(deg_sh.at[rslc], deg_h.at[cid, rslc])


def _edge_agg(logz_s, src_s, dst_s, zeros2d, zeros1d):
    mesh = plsc.VectorSubcoreMesh(core_axis_name="c", subcore_axis_name="s")
    fn = pl.kernel(
        _agg_body,
        out_type=(jax.ShapeDtypeStruct((2, NP, D), jnp.float32),
                  jax.ShapeDtypeStruct((2, NP), jnp.float32)),
        mesh=mesh,
        scratch_types=[
            pltpu.VMEM_SHARED((NP, HD), jnp.float32),
            pltpu.VMEM_SHARED((NP, HD), jnp.float32),
            pltpu.VMEM_SHARED((NP,), jnp.float32),
            pltpu.VMEM((SCH, 128), jnp.int32),
            pltpu.VMEM((SCH, 128), jnp.int32),
            pltpu.VMEM((128, HD), jnp.float32),
            pltpu.VMEM((128, HD), jnp.float32),
            pltpu.VMEM((128, HD), jnp.float32),
            pltpu.VMEM((128, HD), jnp.float32),
            pltpu.VMEM((128,), jnp.float32),
            pltpu.SemaphoreType.DMA,
            pltpu.SemaphoreType.DMA,
            pltpu.SemaphoreType.DMA,
        ],
        compiler_params=pltpu.CompilerParams(use_tc_tiling_on_sc=False),
    )
    return fn(logz_s, src_s, dst_s, zeros2d, zeros1d)


def _dense1_body(f0_ref, f1_ref, f2_ref, w0_ref, w1_ref, w2_ref, b_ref, o_ref):
    hi = lax.Precision.HIGHEST
    v = (jnp.dot(f0_ref[...], w0_ref[0], preferred_element_type=jnp.float32,
                 precision=hi)
         + jnp.dot(f1_ref[...], w1_ref[0], preferred_element_type=jnp.float32,
                   precision=hi)
         + jnp.dot(f2_ref[...], w2_ref[0], preferred_element_type=jnp.float32,
                   precision=hi)
         + b_ref[0])
    o_ref[0] = _log_of_exp(v)


def _dense1(f0, f1, f2, Wi_s, bi_s):
    grid = (2, NP // 1280)
    return pl.pallas_call(
        _dense1_body,
        grid=grid,
        in_specs=[
            pl.BlockSpec((1280, 64), lambda m, r: (r, 0)),
            pl.BlockSpec((1280, 32), lambda m, r: (r, 0)),
            pl.BlockSpec((1280, 32), lambda m, r: (r, 0)),
            pl.BlockSpec((1, 64, D), lambda m, r: (m, 0, 0)),
            pl.BlockSpec((1, 32, D), lambda m, r: (m, 0, 0)),
            pl.BlockSpec((1, 32, D), lambda m, r: (m, 0, 0)),
            pl.BlockSpec((1, 1, D), lambda m, r: (m, 0, 0)),
        ],
        out_specs=pl.BlockSpec((1, 1280, D), lambda m, r: (m, r, 0)),
        out_shape=jax.ShapeDtypeStruct((2, NP, D), jnp.float32),
    )(f0, f1, f2, Wi_s[:, :64], Wi_s[:, 64:96], Wi_s[:, 96:], bi_s)


def _dense2_body(a_ref, l_ref, d_ref, w_ref, b_ref, o_ref):
    a = a_ref[0]
    lz = l_ref[0]
    dg = d_ref[0]
    m = (a + lz) / (dg + 1.0)
    u = _log_of_exp(m)
    z = jnp.dot(u, w_ref[0], preferred_element_type=jnp.float32,
                precision=lax.Precision.HIGHEST) + b_ref[0]
    o_ref[0] = _exp_map(z)


def _dense2(agg_s, logz_s, deg3, Wo_s, bo_s):
    grid = (2, NP // 1280)
    return pl.pallas_call(
        _dense2_body,
        grid=grid,
        in_specs=[
            pl.BlockSpec((1, 1280, D), lambda m, r: (m, r, 0)),
            pl.BlockSpec((1, 1280, D), lambda m, r: (m, r, 0)),
            pl.BlockSpec((1, 1280, 1), lambda m, r: (m, r, 0)),
            pl.BlockSpec((1, D, D), lambda m, r: (m, 0, 0)),
            pl.BlockSpec((1, 1, D), lambda m, r: (m, 0, 0)),
        ],
        out_specs=pl.BlockSpec((1, 1280, D), lambda m, r: (m, r, 0)),
        out_shape=jax.ShapeDtypeStruct((2, NP, D), jnp.float32),
    )(agg_s, logz_s, deg3, Wo_s, bo_s)


def _prep_edges(ei, mode):
    src = ei[0].astype(jnp.int32)
    dst = ei[1].astype(jnp.int32)
    srcp = jnp.concatenate(
        [src, jnp.zeros((EP - E,), jnp.int32)]).reshape(EP // 128, 128)
    dstp = jnp.concatenate(
        [dst, jnp.full((EP - E,), NP - 1, jnp.int32)]).reshape(EP // 128, 128)
    return srcp, dstp


def kernel(x, edge_index_click, edge_index_buy, emb0, emb1, emb2,
           W_in_click, b_in_click, W_out_click, b_out_click,
           W_in_buy, b_in_buy, W_out_buy, b_out_buy):
    xi = jnp.pad(x.astype(jnp.int32), ((0, NP - N), (0, 0)))
    x0, x1, x2 = xi[:, 0], xi[:, 1], xi[:, 2]

    sc, dc = _prep_edges(edge_index_click, 0)
    sb, db = _prep_edges(edge_index_buy, 1)
    src_s = jnp.stack([sc, sb])
    dst_s = jnp.stack([dc, db])

    Wi_s = jnp.stack([W_in_click, W_in_buy])
    bi_s = jnp.stack([b_in_click, b_in_buy]).reshape(2, 1, D)
    Wo_s = jnp.stack([W_out_click, W_out_buy])
    bo_s = jnp.stack([b_out_click, b_out_buy]).reshape(2, 1, D)

    zeros2d = jnp.zeros((ROWS_PT, HD), jnp.float32)
    zeros1d = jnp.zeros((ROWS_PT,), jnp.float32)

    f0, f1, f2 = _gather_feats(x0, x1, x2, emb0, emb1, emb2)
    logz_s = _dense1(f0, f1, f2, Wi_s, bi_s)
    agg_s, deg_s = _edge_agg(logz_s, src_s, dst_s, zeros2d, zeros1d)
    out_s = _dense2(agg_s, logz_s, deg_s.reshape(2, NP, 1), Wo_s, bo_s)
    return (out_s[0, :N], out_s[1, :N])
